# baseline (speedup 1.0000x reference)
import functools

import jax
import jax.numpy as jnp
from jax import lax
from jax.experimental import pallas as pl
from jax.experimental.pallas import tpu as pltpu
from jax.experimental.pallas import tpu_sc as plsc

VOCAB = 1000000
EMB = 32
BATCH = 16384
FIELDS = 26
FPAD = 32

NUM_CORES = 2
NUM_SUBCORES = 16
NUM_WORKERS = NUM_CORES * NUM_SUBCORES
ROWS_PER_WORKER = BATCH // NUM_WORKERS
CHUNK_ROWS = 32
NUM_CHUNKS = ROWS_PER_WORKER // CHUNK_ROWS


def _embedding_gather(ids_pad, table):
  mesh = plsc.VectorSubcoreMesh(core_axis_name="c", subcore_axis_name="s")

  @functools.partial(
      pl.kernel,
      mesh=mesh,
      compiler_params=pltpu.CompilerParams(use_tc_tiling_on_sc=False),
      out_type=jax.ShapeDtypeStruct((BATCH, FIELDS, EMB), jnp.float32),
      scratch_types=[
          pltpu.VMEM((ROWS_PER_WORKER, FPAD), jnp.int32),
          pltpu.VMEM((2, CHUNK_ROWS, FPAD, EMB), jnp.float32),
          pltpu.SemaphoreType.DMA,
          pltpu.SemaphoreType.DMA,
          pltpu.SemaphoreType.DMA,
          pltpu.SemaphoreType.DMA,
      ],
  )
  def k(ids_hbm, table_hbm, out_hbm, idx_v, rows_v, g0, g1, w0, w1):
    wid = lax.axis_index("s") * NUM_CORES + lax.axis_index("c")
    brow = wid * ROWS_PER_WORKER
    gsem = [g0, g1]
    wsem = [w0, w1]

    pltpu.sync_copy(ids_hbm.at[pl.ds(brow, ROWS_PER_WORKER)], idx_v)

    def gather(c):
      b = c % 2

      def row_body(i, carry):
        pltpu.async_copy(
            table_hbm.at[idx_v.at[c * CHUNK_ROWS + i]],
            rows_v.at[b].at[i],
            gsem[b],
        )
        return carry

      lax.fori_loop(0, CHUNK_ROWS, row_body, 0)
      return pltpu.make_async_copy(
          out_hbm.at[pl.ds(0, CHUNK_ROWS)], rows_v.at[b], gsem[b]
      )

    def write(c):
      b = c % 2
      return pltpu.async_copy(
          rows_v.at[b, :, pl.ds(0, FIELDS)],
          out_hbm.at[pl.ds(brow + c * CHUNK_ROWS, CHUNK_ROWS)],
          wsem[b],
      )

    hg = {}
    hw = {}
    hg[0] = gather(0)
    for c in range(NUM_CHUNKS):
      if c + 1 < NUM_CHUNKS:
        if c + 1 >= 2:
          hw[c - 1].wait()
        hg[c + 1] = gather(c + 1)
      hg[c].wait()
      hw[c] = write(c)
    hw[NUM_CHUNKS - 2].wait()
    hw[NUM_CHUNKS - 1].wait()

  return k(ids_pad, table)


def kernel(env_ids, table):
  ids_pad = jnp.pad(
      env_ids.astype(jnp.int32), ((0, 0), (0, FPAD - FIELDS)), mode="edge"
  )
  return _embedding_gather(ids_pad, table)


# speedup vs baseline: 1.0014x; 1.0014x over previous
import functools

import jax
import jax.numpy as jnp
from jax import lax
from jax.experimental import pallas as pl
from jax.experimental.pallas import tpu as pltpu
from jax.experimental.pallas import tpu_sc as plsc

VOCAB = 1000000
EMB = 32
BATCH = 16384
FIELDS = 26
FPAD = 32

NUM_CORES = 2
NUM_SUBCORES = 16
NUM_WORKERS = NUM_CORES * NUM_SUBCORES
ROWS_PER_WORKER = BATCH // NUM_WORKERS
CHUNK_ROWS = 32
NUM_CHUNKS = ROWS_PER_WORKER // CHUNK_ROWS


def _embedding_gather(ids_pad, table):
  mesh = plsc.VectorSubcoreMesh(core_axis_name="c", subcore_axis_name="s")

  @functools.partial(
      pl.kernel,
      mesh=mesh,
      compiler_params=pltpu.CompilerParams(use_tc_tiling_on_sc=False),
      out_type=jax.ShapeDtypeStruct((BATCH, FIELDS, EMB), jnp.float32),
      scratch_types=[
          pltpu.VMEM((ROWS_PER_WORKER, FPAD), jnp.int32),
          pltpu.VMEM((2, CHUNK_ROWS, FPAD, EMB), jnp.float32),
          pltpu.SemaphoreType.DMA,
          pltpu.SemaphoreType.DMA,
          pltpu.SemaphoreType.DMA,
          pltpu.SemaphoreType.DMA,
      ],
  )
  def k(ids_hbm, table_hbm, out_hbm, idx_v, rows_v, g0, g1, w0, w1):
    wid = lax.axis_index("s") * NUM_CORES + lax.axis_index("c")
    brow = wid * ROWS_PER_WORKER
    gsem = [g0, g1]
    wsem = [w0, w1]

    pltpu.sync_copy(ids_hbm.at[pl.ds(brow, ROWS_PER_WORKER)], idx_v)

    def gather(c):
      b = c % 2

      def row_body(i, carry):
        pltpu.async_copy(
            table_hbm.at[idx_v.at[c * CHUNK_ROWS + i]],
            rows_v.at[b].at[i],
            gsem[b],
        )
        return carry

      lax.fori_loop(0, CHUNK_ROWS, row_body, 0)
      return pltpu.make_async_copy(
          out_hbm.at[pl.ds(0, CHUNK_ROWS)], rows_v.at[b], gsem[b]
      )

    def write(c):
      b = c % 2
      return pltpu.async_copy(
          rows_v.at[b, :, pl.ds(0, FIELDS)],
          out_hbm.at[pl.ds(brow + c * CHUNK_ROWS, CHUNK_ROWS)],
          wsem[b],
      )

    hg = {}
    hw = {}
    hg[0] = gather(0)
    for c in range(NUM_CHUNKS):
      if c + 1 < NUM_CHUNKS:
        if c + 1 >= 2:
          hw[c - 1].wait()
        hg[c + 1] = gather(c + 1)
      hg[c].wait()
      hw[c] = write(c)
    hw[NUM_CHUNKS - 2].wait()
    hw[NUM_CHUNKS - 1].wait()

  return k(ids_pad, table)


def kernel(env_ids, table):
  ids_pad = jnp.pad(
      env_ids.astype(jnp.int32), ((0, 0), (0, FPAD - FIELDS)), mode="edge"
  )
  ids_pad = jax.lax.optimization_barrier(ids_pad)
  return _embedding_gather(ids_pad, table)


# speedup vs baseline: 1.0058x; 1.0043x over previous
import functools

import jax
import jax.numpy as jnp
from jax import lax
from jax.experimental import pallas as pl
from jax.experimental.pallas import tpu as pltpu
from jax.experimental.pallas import tpu_sc as plsc

VOCAB = 1000000
EMB = 32
BATCH = 16384
FIELDS = 26
FPAD = 32

NUM_CORES = 2
NUM_SUBCORES = 16
NUM_WORKERS = NUM_CORES * NUM_SUBCORES
ROWS_PER_WORKER = BATCH // NUM_WORKERS
CHUNK_ROWS = 32
NUM_CHUNKS = ROWS_PER_WORKER // CHUNK_ROWS

_MESH = dict(core_axis_name="c", subcore_axis_name="s")


def _ids_format(env_ids):
  @functools.partial(
      pl.kernel,
      mesh=plsc.VectorSubcoreMesh(**_MESH),
      compiler_params=pltpu.CompilerParams(needs_layout_passes=False),
      out_type=jax.ShapeDtypeStruct((BATCH * FPAD,), jnp.int32),
      scratch_types=[
          pltpu.VMEM((ROWS_PER_WORKER, FIELDS), jnp.int32),
          pltpu.VMEM((ROWS_PER_WORKER * FPAD,), jnp.int32),
      ],
  )
  def k(ids_hbm, out_hbm, ids_v, flat_v):
    wid = lax.axis_index("s") * NUM_CORES + lax.axis_index("c")
    brow = wid * ROWS_PER_WORKER
    pltpu.sync_copy(ids_hbm.at[pl.ds(brow, ROWS_PER_WORKER)], ids_v)
    lane = lax.iota(jnp.int32, 16)
    hi_col = jnp.minimum(lane + 16, FIELDS - 1)

    def row_body(r, carry):
      lo = ids_v.at[r][pl.ds(0, 16)]
      rb = jnp.full((16,), r, jnp.int32)
      hi = plsc.load_gather(ids_v, [rb, hi_col])
      flat_v[pl.ds(r * FPAD, 16)] = lo
      flat_v[pl.ds(r * FPAD + 16, 16)] = hi
      return carry

    lax.fori_loop(0, ROWS_PER_WORKER, row_body, 0)
    pltpu.sync_copy(
        flat_v, out_hbm.at[pl.ds(brow * FPAD, ROWS_PER_WORKER * FPAD)]
    )

  return k(env_ids)


def _embedding_gather(idx_flat, table):
  @functools.partial(
      pl.kernel,
      mesh=plsc.VectorSubcoreMesh(**_MESH),
      compiler_params=pltpu.CompilerParams(use_tc_tiling_on_sc=False),
      out_type=jax.ShapeDtypeStruct((BATCH, FIELDS, EMB), jnp.float32),
      scratch_types=[
          pltpu.VMEM((ROWS_PER_WORKER * FPAD,), jnp.int32),
          pltpu.VMEM((2, CHUNK_ROWS, FIELDS, EMB), jnp.float32),
          pltpu.SemaphoreType.DMA,
          pltpu.SemaphoreType.DMA,
          pltpu.SemaphoreType.DMA,
          pltpu.SemaphoreType.DMA,
      ],
  )
  def k(idx_hbm, table_hbm, out_hbm, idx_v, rows_v, g0, g1, w0, w1):
    wid = lax.axis_index("s") * NUM_CORES + lax.axis_index("c")
    brow = wid * ROWS_PER_WORKER
    gsem = [g0, g1]
    wsem = [w0, w1]

    pltpu.sync_copy(
        idx_hbm.at[pl.ds(brow * FPAD, ROWS_PER_WORKER * FPAD)], idx_v
    )

    def gather(c):
      b = c % 2

      def row_body(i, carry):
        pltpu.async_copy(
            table_hbm.at[idx_v.at[pl.ds((c * CHUNK_ROWS + i) * FPAD, FIELDS)]],
            rows_v.at[b].at[i],
            gsem[b],
        )
        return carry

      lax.fori_loop(0, CHUNK_ROWS, row_body, 0)
      return pltpu.make_async_copy(
          out_hbm.at[pl.ds(0, CHUNK_ROWS)], rows_v.at[b], gsem[b]
      )

    def write(c):
      b = c % 2
      return pltpu.async_copy(
          rows_v.at[b],
          out_hbm.at[pl.ds(brow + c * CHUNK_ROWS, CHUNK_ROWS)],
          wsem[b],
      )

    hg = {}
    hw = {}
    hg[0] = gather(0)
    for c in range(NUM_CHUNKS):
      if c + 1 < NUM_CHUNKS:
        if c + 1 >= 2:
          hw[c - 1].wait()
        hg[c + 1] = gather(c + 1)
      hg[c].wait()
      hw[c] = write(c)
    hw[NUM_CHUNKS - 2].wait()
    hw[NUM_CHUNKS - 1].wait()

  return k(idx_flat, table)


def kernel(env_ids, table):
  idx_flat = _ids_format(env_ids.astype(jnp.int32))
  return _embedding_gather(idx_flat, table)
